# baseline (speedup 1.0000x reference)
import functools

import jax
import jax.numpy as jnp
from jax.experimental import pallas as pl
from jax.experimental.pallas import tpu as pltpu

_E = 8
_C = 10
_BB = 8
_BBT = 4
_BBC = 128


def _roll_rows(a, s):
    n = a.shape[0]
    s = s % n
    if s == 0:
        return a
    return jnp.concatenate([a[s:], a[:s]], axis=0)


def _im2col(a, bb, h, w, c):
    nrows = bb * h * w
    idx = jax.lax.broadcasted_iota(jnp.int32, (nrows, 1), 0)
    yy = (idx // w) % h
    xx = idx % w
    pieces = []
    for dy in (-1, 0, 1):
        vy = jnp.logical_and(yy + dy >= 0, yy + dy < h)
        for dx in (-1, 0, 1):
            vx = jnp.logical_and(xx + dx >= 0, xx + dx < w)
            piece = _roll_rows(a, dy * w + dx)
            pieces.append(jnp.where(jnp.logical_and(vy, vx), piece, 0.0))
    return jnp.concatenate(pieces, axis=1)


def _maxpool_rows(a, bb, h, w, c):
    a = a.reshape(bb * h * (w // 2), 2, c).max(axis=1)
    a = a.reshape(bb * (h // 2), 2, (w // 2), c).max(axis=1)
    return a.reshape(bb * (h // 2) * (w // 2), c)


def _expert_kernel(xcol_ref, w1, b1, w2, b2, w3, b3, w4, b4, w5, b5,
                   fw, fb, cw, cb, out_ref):
    bb = _BB
    xcol = xcol_ref[...]
    a1 = jnp.maximum(jnp.dot(xcol, w1[0], preferred_element_type=jnp.float32)
                     + b1[0], 0.0)
    a1 = _maxpool_rows(a1, bb, 32, 32, 32)
    col = _im2col(a1, bb, 16, 16, 32)
    a2 = jnp.maximum(jnp.dot(col, w2[0], preferred_element_type=jnp.float32)
                     + b2[0], 0.0)
    col = _im2col(a2, bb, 16, 16, 64)
    a3 = jnp.maximum(jnp.dot(col, w3[0], preferred_element_type=jnp.float32)
                     + b3[0], 0.0)
    a3 = _maxpool_rows(a3, bb, 16, 16, 64)
    col = _im2col(a3, bb, 8, 8, 64)
    a4 = jnp.maximum(jnp.dot(col, w4[0], preferred_element_type=jnp.float32)
                     + b4[0], 0.0)
    col = _im2col(a4, bb, 8, 8, 128)
    a5 = jnp.maximum(jnp.dot(col, w5[0], preferred_element_type=jnp.float32)
                     + b5[0], 0.0)
    f = jnp.mean(a5.reshape(bb, 64, 128), axis=1)
    f = jnp.maximum(jnp.dot(f, fw[0], preferred_element_type=jnp.float32)
                    + fb[0], 0.0)
    lg = jnp.dot(f, cw[0], preferred_element_type=jnp.float32) + cb[0]
    m = jnp.max(lg, axis=1, keepdims=True)
    ex = jnp.exp(lg - m)
    p = ex / jnp.sum(ex, axis=1, keepdims=True)
    conf = jnp.sum(p * jnp.log(p + 1e-12), axis=1, keepdims=True)
    out_ref[0, 0] = jnp.concatenate(
        [lg, conf, jnp.zeros((bb, 5), jnp.float32)], axis=1)


def _trunk_kernel(xcol_ref, w1_r, b1_r, w2_r, b2_r, tw_r, twb_r, gw1_r, gb1_r,
                  gw2_r, gb2_r, out_ref):
    bb = _BBT
    xcol = xcol_ref[...]
    w1, b1, w2, b2 = w1_r[...], b1_r[...], w2_r[...], b2_r[...]
    tw, twb = tw_r[...], twb_r[...]
    gw1, gb1, gw2, gb2 = gw1_r[...], gb1_r[...], gw2_r[...], gb2_r[...]
    a1 = jnp.maximum(jnp.dot(xcol, w1, preferred_element_type=jnp.float32)
                     + b1, 0.0)
    col = _im2col(a1, bb, 32, 32, 32)
    a2 = jnp.maximum(jnp.dot(col, w2, preferred_element_type=jnp.float32)
                     + b2, 0.0)
    a2 = _maxpool_rows(a2, bb, 32, 32, 32)
    a2 = jnp.mean(a2.reshape(bb * 16 * 2, 8, 32), axis=1)
    a2 = jnp.mean(a2.reshape(bb, 2, 8, 2, 32), axis=2)
    rf = a2.reshape(bb, 128)
    rf = jnp.maximum(jnp.dot(rf, tw, preferred_element_type=jnp.float32)
                     + twb, 0.0)
    hg = jnp.maximum(jnp.dot(rf, gw1, preferred_element_type=jnp.float32)
                     + gb1, 0.0)
    out_ref[0] = jnp.dot(hg, gw2, preferred_element_type=jnp.float32) + gb2


def _combine_kernel(eo_ref, sc_ref, out_ref):
    bb = sc_ref.shape[0]
    eo = eo_ref[...]
    lg = eo[:, :, 0:10]
    conf = jnp.transpose(eo[:, :, 10])
    combined = 0.7 * sc_ref[...] + 0.3 * conf - 0.25
    lane = jax.lax.broadcasted_iota(jnp.int32, (bb, _E), 1)
    m1 = jnp.max(combined, axis=1, keepdims=True)
    i1 = jnp.min(jnp.where(combined == m1, lane, _E), axis=1, keepdims=True)
    mask1 = lane == i1
    rest = jnp.where(mask1, -1e30, combined)
    m2 = jnp.max(rest, axis=1, keepdims=True)
    i2 = jnp.min(jnp.where(rest == m2, lane, _E), axis=1, keepdims=True)
    mask2 = lane == i2
    g1 = 1.0 / (1.0 + jnp.exp(m2 - m1))
    wts = g1 * mask1 + (1.0 - g1) * mask2
    acc = jnp.zeros((bb, 10), jnp.float32)
    for e in range(_E):
        acc = acc + wts[:, e:e + 1] * lg[e]
    out_ref[...] = acc


def _fold_bn(w, g, b, cb=None):
    s = g / jnp.sqrt(1.0 + 1e-5)
    co, ci = w.shape[0], w.shape[1]
    wf = jnp.transpose(w, (2, 3, 1, 0)).reshape(9 * ci, co) * s[None, :]
    bias = b if cb is None else cb * s + b
    return wf, bias.reshape(1, -1)


def _conv1_cols(x):
    xp = jnp.pad(x, ((0, 0), (0, 0), (1, 1), (1, 1)))
    cols = jnp.stack([xp[:, :, kh:kh + 32, kw:kw + 32]
                      for kh in range(3) for kw in range(3)], axis=1)
    cols = jnp.transpose(cols, (0, 3, 4, 1, 2))
    cols = cols.reshape(x.shape[0] * 1024, 27)
    return jnp.pad(cols, ((0, 0), (0, 5)))


@jax.jit
def kernel(x, params):
    p = params
    ep = p['experts']
    batch = x.shape[0]
    nb = batch // _BB
    nbt = batch // _BBT

    xcol = _conv1_cols(x)

    def fold_e(w, g, b, cb):
        s = (g / jnp.sqrt(1.0 + 1e-5))
        co, ci = w.shape[1], w.shape[2]
        wf = jnp.transpose(w, (0, 3, 4, 2, 1)).reshape(_E, 9 * ci, co)
        wf = wf * s[:, None, :]
        return wf, (cb * s + b).reshape(_E, 1, co)

    w1, b1 = fold_e(ep['c1'], ep['g1'], ep['b1'], ep['c1b'])
    w1 = jnp.pad(w1, ((0, 0), (0, 5), (0, 0)))
    w2, b2 = fold_e(ep['c2'], ep['g2'], ep['b2'], ep['c2b'])
    w3, b3 = fold_e(ep['c3'], ep['g3'], ep['b3'], ep['c3b'])
    w4, b4 = fold_e(ep['c4'], ep['g4'], ep['b4'], ep['c4b'])
    w5, b5 = fold_e(ep['c5'], ep['g5'], ep['b5'], ep['c5b'])
    fw = ep['fw']
    fb = ep['fb'].reshape(_E, 1, 128)
    cw = ep['cw']
    cb = ep['cb'].reshape(_E, 1, 10)

    def wspec(k, c):
        return pl.BlockSpec((1, k, c), lambda e, b: (e, 0, 0))

    eo = pl.pallas_call(
        _expert_kernel,
        grid=(_E, nb),
        in_specs=[
            pl.BlockSpec((_BB * 1024, 32), lambda e, b: (b, 0)),
            wspec(32, 32), wspec(1, 32),
            wspec(288, 64), wspec(1, 64),
            wspec(576, 64), wspec(1, 64),
            wspec(576, 128), wspec(1, 128),
            wspec(1152, 128), wspec(1, 128),
            wspec(128, 128), wspec(1, 128),
            wspec(128, 10), wspec(1, 10),
        ],
        out_specs=pl.BlockSpec((1, 1, _BB, 16), lambda e, b: (e, b, 0, 0)),
        out_shape=jax.ShapeDtypeStruct((_E, nb, _BB, 16), jnp.float32),
        compiler_params=pltpu.CompilerParams(
            dimension_semantics=("parallel", "parallel")),
    )(xcol, w1, b1, w2, b2, w3, b3, w4, b4, w5, b5, fw, fb, cw, cb)
    eo = eo.reshape(_E, batch, 16)

    tw1, tb1 = _fold_bn(p['t_c1'], p['t_g1'], p['t_b1'])
    tw1 = jnp.pad(tw1, ((0, 5), (0, 0)))
    tw2, tb2 = _fold_bn(p['t_c2'], p['t_g2'], p['t_b2'])
    twp = jnp.transpose(p['t_w'].reshape(32, 4, 64), (1, 0, 2)).reshape(128, 64)

    def full(a):
        return pl.BlockSpec(a.shape, lambda b: (0,) * a.ndim)

    twb = p['t_wb'].reshape(1, 64)
    gw1 = p['g_w1']
    gb1 = p['g_b1'].reshape(1, 32)
    gw2 = p['g_w2']
    gb2 = p['g_b2'].reshape(1, 8)
    scores = pl.pallas_call(
        _trunk_kernel,
        grid=(nbt,),
        in_specs=[
            pl.BlockSpec((_BBT * 1024, 32), lambda b: (b, 0)),
            full(tw1), full(tb1), full(tw2), full(tb2),
            full(twp), full(twb), full(gw1), full(gb1), full(gw2), full(gb2),
        ],
        out_specs=pl.BlockSpec((1, _BBT, 8), lambda b: (b, 0, 0)),
        out_shape=jax.ShapeDtypeStruct((nbt, _BBT, 8), jnp.float32),
        compiler_params=pltpu.CompilerParams(
            dimension_semantics=("parallel",)),
    )(xcol, tw1, tb1, tw2, tb2, twp, twb, gw1, gb1, gw2, gb2)
    scores = scores.reshape(batch, 8)

    bbc = min(_BBC, batch)
    out = pl.pallas_call(
        _combine_kernel,
        grid=(batch // bbc,),
        in_specs=[
            pl.BlockSpec((_E, bbc, 16), lambda b: (0, b, 0)),
            pl.BlockSpec((bbc, 8), lambda b: (b, 0)),
        ],
        out_specs=pl.BlockSpec((bbc, 10), lambda b: (b, 0)),
        out_shape=jax.ShapeDtypeStruct((batch, 10), jnp.float32),
        compiler_params=pltpu.CompilerParams(
            dimension_semantics=("arbitrary",)),
    )(eo, scores)
    return out
